# baseline (speedup 1.0000x reference)
import functools

import jax
import jax.numpy as jnp
from jax import lax
from jax.experimental import pallas as pl
from jax.experimental.pallas import tpu as pltpu
from jax.experimental.pallas import tpu_sc as plsc

B, N, T, S = 1024, 148, 64, 32
DIM = 64

BB = 64
NW = 32
BPW = B // NW
_CHUNKS = tuple(range(0, 144, 16)) + (132,)


def _elu(x):
    return jnp.where(x > 0, x, jnp.exp(x) - 1.0)


def _sc_pool_body(y_hbm, mn_hbm, mx_hbm, y_v, mn_row, mx_row):
    wid = lax.axis_index("s") * 2 + lax.axis_index("c")
    base = wid * BPW
    pinf = jnp.full((16,), jnp.inf, jnp.float32)
    ninf = jnp.full((16,), -jnp.inf, jnp.float32)

    def batch_body(b, carry):
        bidx = base + b
        pltpu.sync_copy(y_hbm.at[bidx], y_v)
        for n0 in _CHUNKS:
            mnc, mxc = pinf, ninf
            for s in range(S):
                v = y_v[s, pl.ds(n0, 16)]
                mnc = jnp.minimum(mnc, jnp.where(v == 0.0, 100.0, v))
                mxc = jnp.maximum(mxc, v)
            mn_row[pl.ds(n0, 16)] = mnc
            mx_row[pl.ds(n0, 16)] = mxc
        pltpu.sync_copy(mn_row, mn_hbm.at[bidx])
        pltpu.sync_copy(mx_row, mx_hbm.at[bidx])
        return carry

    lax.fori_loop(0, BPW, batch_body, 0)


_sc_pool = functools.partial(
    pl.kernel,
    mesh=plsc.VectorSubcoreMesh(core_axis_name="c", subcore_axis_name="s"),
    out_type=[jax.ShapeDtypeStruct((B, N), jnp.float32),
              jax.ShapeDtypeStruct((B, N), jnp.float32)],
    scratch_types=[pltpu.VMEM((S, N), jnp.float32),
                   pltpu.VMEM((N,), jnp.float32),
                   pltpu.VMEM((N,), jnp.float32)],
)(_sc_pool_body)


def _tc_kernel(phi_ref, mn_ref, mx_ref, g_ref, W1_ref, b1_ref, W2_ref, b2_ref,
               C1_ref, cb1_ref, C2_ref, cb2_ref, C3_ref, cb3_ref, out_ref):
    phi = phi_ref[...]
    g = g_ref[...]

    mn_phi = jnp.min(jnp.where(phi == 0.0, 100.0, phi), axis=2,
                     keepdims=True)
    mx_phi = jnp.max(phi, axis=2, keepdims=True)
    mn3 = jnp.minimum(mn_phi, mn_ref[...][:, :, None])
    mx3 = jnp.maximum(mx_phi, mx_ref[...][:, :, None])

    deg3 = jnp.sum(g, axis=2, keepdims=True) + 1.0
    dinv3 = lax.rsqrt(deg3)

    r2 = jnp.concatenate([mn3, mx3], axis=2) * dinv3
    u1 = lax.dot_general(g, r2, (((2,), (1,)), ((0,), (0,))),
                         preferred_element_type=jnp.float32) + r2
    p_mn = u1[:, :, 0:1] * dinv3
    p_mx = u1[:, :, 1:2] * dinv3
    W1r0 = W1_ref[0:1, :]
    W1r1 = W1_ref[1:2, :]
    h1 = _elu(p_mn * W1r0[None] + p_mx * W1r1[None]
              + b1_ref[...][None])

    hs = h1 * dinv3
    u = lax.dot_general(g, hs, (((2,), (1,)), ((0,), (0,))),
                        preferred_element_type=jnp.float32) + hs
    x2 = u * dinv3
    t2 = lax.dot_general(x2, W2_ref[...], (((2,), (0,)), ((), ())),
                         preferred_element_type=jnp.float32)
    h2 = _elu(t2 + b2_ref[...][None])
    pooled = jnp.sum(h2, axis=1) * (1.0 / N)

    z = _elu(jnp.dot(pooled, C1_ref[...], preferred_element_type=jnp.float32)
             + cb1_ref[...])
    z = _elu(jnp.dot(z, C2_ref[...], preferred_element_type=jnp.float32)
             + cb2_ref[...])
    out_ref[...] = (jnp.dot(z, C3_ref[...], preferred_element_type=jnp.float32)
                    + cb3_ref[...])


def _tc_call(phi, mn, mx, g, W1, b1r, W2, b2r, C1, cb1r, C2, cb2r, C3, cb3r):
    wspec = lambda shape: pl.BlockSpec(shape, lambda i: (0,) * len(shape))
    return pl.pallas_call(
        _tc_kernel,
        grid=(B // BB,),
        in_specs=[
            pl.BlockSpec((BB, N, T), lambda i: (i, 0, 0)),
            pl.BlockSpec((BB, N), lambda i: (i, 0)),
            pl.BlockSpec((BB, N), lambda i: (i, 0)),
            pl.BlockSpec((BB, N, N), lambda i: (i, 0, 0)),
            wspec(W1.shape),
            wspec(b1r.shape),
            wspec(W2.shape),
            wspec(b2r.shape),
            wspec(C1.shape),
            wspec(cb1r.shape),
            wspec(C2.shape),
            wspec(cb2r.shape),
            wspec(C3.shape),
            wspec(cb3r.shape),
        ],
        out_specs=pl.BlockSpec((BB, 2), lambda i: (i, 0)),
        out_shape=jax.ShapeDtypeStruct((B, 2), jnp.float32),
    )(phi, mn, mx, g, W1, b1r, W2, b2r, C1, cb1r, C2, cb2r, C3, cb3r)


@jax.jit
def kernel(phi, y, g, W1, b1, W2, b2, C1, cb1, C2, cb2, C3, cb3):
    mn, mx = _sc_pool(y)
    return _tc_call(phi, mn, mx, g, W1, b1.reshape(1, -1), W2,
                    b2.reshape(1, -1), C1, cb1.reshape(1, -1), C2,
                    cb2.reshape(1, -1), C3, cb3.reshape(1, -1))


# speedup vs baseline: 1.0055x; 1.0055x over previous
import functools

import jax
import jax.numpy as jnp
from jax import lax
from jax.experimental import pallas as pl
from jax.experimental.pallas import tpu as pltpu
from jax.experimental.pallas import tpu_sc as plsc

B, N, T, S = 1024, 148, 64, 32
DIM = 64

BB = 32
BP = 128
NW = 32
BPW = B // NW
_CHUNKS = tuple(range(0, 144, 16)) + (132,)


def _elu(x):
    return jnp.where(x > 0, x, jnp.exp(x) - 1.0)


def _sc_pool_body(y_hbm, mn_hbm, mx_hbm, y_v, mn_row, mx_row):
    wid = lax.axis_index("s") * 2 + lax.axis_index("c")
    base = wid * BPW
    pinf = jnp.full((16,), jnp.inf, jnp.float32)
    ninf = jnp.full((16,), -jnp.inf, jnp.float32)

    def batch_body(b, carry):
        bidx = base + b
        pltpu.sync_copy(y_hbm.at[bidx], y_v)
        for n0 in _CHUNKS:
            mnc, mxc = pinf, ninf
            for s in range(S):
                v = y_v[s, pl.ds(n0, 16)]
                mnc = jnp.minimum(mnc, jnp.where(v == 0.0, 100.0, v))
                mxc = jnp.maximum(mxc, v)
            mn_row[pl.ds(n0, 16)] = mnc
            mx_row[pl.ds(n0, 16)] = mxc
        pltpu.sync_copy(mn_row, mn_hbm.at[bidx])
        pltpu.sync_copy(mx_row, mx_hbm.at[bidx])
        return carry

    lax.fori_loop(0, BPW, batch_body, 0)


_sc_pool = functools.partial(
    pl.kernel,
    mesh=plsc.VectorSubcoreMesh(core_axis_name="c", subcore_axis_name="s"),
    out_type=[jax.ShapeDtypeStruct((B, N), jnp.float32),
              jax.ShapeDtypeStruct((B, N), jnp.float32)],
    scratch_types=[pltpu.VMEM((S, N), jnp.float32),
                   pltpu.VMEM((N,), jnp.float32),
                   pltpu.VMEM((N,), jnp.float32)],
)(_sc_pool_body)


def _pool_kernel(phi_ref, mn_ref, mx_ref):
    phi = phi_ref[...]
    mn_ref[...] = jnp.min(jnp.where(phi == 0.0, 100.0, phi), axis=2)
    mx_ref[...] = jnp.max(phi, axis=2)


def _pool_call(phi):
    return pl.pallas_call(
        _pool_kernel,
        grid=(B // BP,),
        in_specs=[pl.BlockSpec((BP, N, T), lambda i: (i, 0, 0))],
        out_specs=[pl.BlockSpec((BP, N), lambda i: (i, 0)),
                   pl.BlockSpec((BP, N), lambda i: (i, 0))],
        out_shape=[jax.ShapeDtypeStruct((B, N), jnp.float32),
                   jax.ShapeDtypeStruct((B, N), jnp.float32)],
    )(phi)


def _tc_kernel(mnp_ref, mxp_ref, mn_ref, mx_ref, g_ref, W1_ref, b1_ref,
               W2_ref, b2_ref, C1_ref, cb1_ref, C2_ref, cb2_ref, C3_ref,
               cb3_ref, out_ref):
    g = g_ref[...]

    mn3 = jnp.minimum(mnp_ref[...], mn_ref[...])[:, :, None]
    mx3 = jnp.maximum(mxp_ref[...], mx_ref[...])[:, :, None]

    deg3 = jnp.sum(g, axis=2, keepdims=True) + 1.0
    dinv3 = lax.rsqrt(deg3)

    r2 = jnp.concatenate([mn3, mx3], axis=2) * dinv3
    u1 = lax.dot_general(g, r2, (((2,), (1,)), ((0,), (0,))),
                         preferred_element_type=jnp.float32) + r2
    p_mn = u1[:, :, 0:1] * dinv3
    p_mx = u1[:, :, 1:2] * dinv3
    W1r0 = W1_ref[0:1, :]
    W1r1 = W1_ref[1:2, :]
    h1 = _elu(p_mn * W1r0[None] + p_mx * W1r1[None]
              + b1_ref[...][None])

    hs = h1 * dinv3
    u = lax.dot_general(g, hs, (((2,), (1,)), ((0,), (0,))),
                        preferred_element_type=jnp.float32) + hs
    x2 = u * dinv3
    t2 = lax.dot_general(x2, W2_ref[...], (((2,), (0,)), ((), ())),
                         preferred_element_type=jnp.float32)
    h2 = _elu(t2 + b2_ref[...][None])
    pooled = jnp.sum(h2, axis=1) * (1.0 / N)

    z = _elu(jnp.dot(pooled, C1_ref[...], preferred_element_type=jnp.float32)
             + cb1_ref[...])
    z = _elu(jnp.dot(z, C2_ref[...], preferred_element_type=jnp.float32)
             + cb2_ref[...])
    out_ref[...] = (jnp.dot(z, C3_ref[...], preferred_element_type=jnp.float32)
                    + cb3_ref[...])


def _tc_call(mnp, mxp, mn, mx, g, W1, b1r, W2, b2r, C1, cb1r, C2, cb2r, C3,
             cb3r):
    wspec = lambda shape: pl.BlockSpec(shape, lambda i: (0,) * len(shape))
    return pl.pallas_call(
        _tc_kernel,
        grid=(B // BB,),
        in_specs=[
            pl.BlockSpec((BB, N), lambda i: (i, 0)),
            pl.BlockSpec((BB, N), lambda i: (i, 0)),
            pl.BlockSpec((BB, N), lambda i: (i, 0)),
            pl.BlockSpec((BB, N), lambda i: (i, 0)),
            pl.BlockSpec((BB, N, N), lambda i: (i, 0, 0)),
            wspec(W1.shape),
            wspec(b1r.shape),
            wspec(W2.shape),
            wspec(b2r.shape),
            wspec(C1.shape),
            wspec(cb1r.shape),
            wspec(C2.shape),
            wspec(cb2r.shape),
            wspec(C3.shape),
            wspec(cb3r.shape),
        ],
        out_specs=pl.BlockSpec((BB, 2), lambda i: (i, 0)),
        out_shape=jax.ShapeDtypeStruct((B, 2), jnp.float32),
    )(mnp, mxp, mn, mx, g, W1, b1r, W2, b2r, C1, cb1r, C2, cb2r, C3, cb3r)


@jax.jit
def kernel(phi, y, g, W1, b1, W2, b2, C1, cb1, C2, cb2, C3, cb3):
    mn, mx = _sc_pool(y)
    mnp, mxp = _pool_call(phi)
    return _tc_call(mnp, mxp, mn, mx, g, W1, b1.reshape(1, -1), W2,
                    b2.reshape(1, -1), C1, cb1.reshape(1, -1), C2,
                    cb2.reshape(1, -1), C3, cb3.reshape(1, -1))
